# baseline (speedup 1.0000x reference)
import functools

import jax
import jax.numpy as jnp
from jax import lax
from jax.experimental import pallas as pl
from jax.experimental.pallas import tpu as pltpu
from jax.experimental.pallas import tpu_sc as plsc

N = 8192
M = 2048
E = 8
C = 1024
TRASH = N
DISP_ROWS = (E + 1) * C

NW = 32
TPW = N // NW
RCH = 16
NBUF = 3

TB = 1024
GRID_G = N // TB
NBLK = 1024


def _gate_kernel(tok_ref, wgt_ref, sidx_ref, g_ref, laux_ref,
                 counts_ref, me_ref, tril_ref):
    i = pl.program_id(0)

    @pl.when(i == 0)
    def _init():
        counts_ref[...] = jnp.zeros_like(counts_ref)
        me_ref[...] = jnp.zeros_like(me_ref)
        r_i = lax.broadcasted_iota(jnp.int32, (TB, TB), 0)
        c_i = lax.broadcasted_iota(jnp.int32, (TB, TB), 1)
        tril_ref[...] = (r_i >= c_i).astype(jnp.float32)

    tok = tok_ref[...]
    logits = jnp.dot(tok, wgt_ref[...],
                     preferred_element_type=jnp.float32)
    m = jnp.max(logits, axis=1, keepdims=True)
    ex = jnp.exp(logits - m)
    s = jnp.sum(ex, axis=1, keepdims=True)
    gates = ex / s
    iota_e = lax.broadcasted_iota(jnp.int32, (TB, E), 1)
    idxv = jnp.min(jnp.where(logits == m, iota_e, E), axis=1, keepdims=True)
    mask1 = (iota_e == idxv).astype(jnp.float32)
    gates1 = jnp.sum(gates * mask1, axis=1, keepdims=True)

    loc_incl = jnp.dot(tril_ref[...], mask1,
                       preferred_element_type=jnp.float32)
    loc = loc_incl - 1.0 + counts_ref[...]
    loc_tok = jnp.sum(loc * mask1, axis=1, keepdims=True).astype(jnp.int32)

    keep = loc_tok < C
    flat = idxv * C + loc_tok
    sidx_ref[0] = jnp.where(keep, flat, TRASH)
    g_ref[0] = jnp.where(keep, gates1, 0.0)

    counts_ref[...] += jnp.sum(mask1, axis=0, keepdims=True)
    me_ref[...] += jnp.sum(gates, axis=0, keepdims=True)

    @pl.when(i == GRID_G - 1)
    def _fin():
        laux_ref[...] = (jnp.sum(me_ref[...] * counts_ref[...],
                                 keepdims=True)
                         * (E / (N * N)))


_gate_call = pl.pallas_call(
    _gate_kernel,
    grid=(GRID_G,),
    in_specs=[
        pl.BlockSpec((TB, M), lambda i: (i, 0)),
        pl.BlockSpec((M, E), lambda i: (0, 0)),
    ],
    out_specs=[
        pl.BlockSpec((1, TB, 1), lambda i: (i, 0, 0)),
        pl.BlockSpec((1, TB, 1), lambda i: (i, 0, 0)),
        pl.BlockSpec((1, 1), lambda i: (0, 0)),
    ],
    out_shape=[
        jax.ShapeDtypeStruct((GRID_G, TB, 1), jnp.int32),
        jax.ShapeDtypeStruct((GRID_G, TB, 1), jnp.float32),
        jax.ShapeDtypeStruct((1, 1), jnp.float32),
    ],
    scratch_shapes=[
        pltpu.VMEM((1, E), jnp.float32),
        pltpu.VMEM((1, E), jnp.float32),
        pltpu.VMEM((TB, TB), jnp.float32),
    ],
    compiler_params=pltpu.CompilerParams(
        dimension_semantics=("arbitrary",)),
)


_sc_mesh = plsc.VectorSubcoreMesh(core_axis_name="c", subcore_axis_name="s")


@functools.partial(
    pl.kernel,
    mesh=_sc_mesh,
    out_type=[
        jax.ShapeDtypeStruct((DISP_ROWS, M), jnp.float32),
        jax.ShapeDtypeStruct((DISP_ROWS, 128), jnp.float32),
    ],
    scratch_types=[
        pltpu.VMEM((TPW,), jnp.int32),
        pltpu.VMEM((TPW,), jnp.float32),
        pltpu.VMEM((NBUF, RCH, M), jnp.float32),
        pltpu.VMEM((NBUF, RCH, 128), jnp.float32),
        pltpu.SemaphoreType.DMA,
        pltpu.SemaphoreType.DMA,
        pltpu.SemaphoreType.DMA,
        pltpu.SemaphoreType.DMA,
        pltpu.SemaphoreType.DMA,
        pltpu.SemaphoreType.DMA,
        pltpu.SemaphoreType.DMA,
        pltpu.SemaphoreType.DMA,
        pltpu.SemaphoreType.DMA,
    ],
)
def _dispatch(tok_hbm, sidx_hbm, g_hbm, disp_hbm, gslot_hbm,
              idx_v, g_v, rows_v, gstage_v,
              l0, l1, l2, s0, s1, s2, q0, q1, q2):
    wid = lax.axis_index("s") * 2 + lax.axis_index("c")
    base = wid * TPW
    pltpu.sync_copy(sidx_hbm.at[pl.ds(base, TPW)], idx_v)
    pltpu.sync_copy(g_hbm.at[pl.ds(base, TPW)], g_v)
    lsems, ssems, qsems = [l0, l1, l2], [s0, s1, s2], [q0, q1, q2]
    NCH = TPW // RCH

    def load(j):
        return pltpu.async_copy(tok_hbm.at[pl.ds(base + j * RCH, RCH)],
                                rows_v.at[j % NBUF], lsems[j % NBUF])

    loads = [load(0)] + [None] * (NCH - 1)
    scats = [None] * NCH
    gscats = [None] * NCH
    for j in range(NCH):
        b = j % NBUF
        if j + 1 < NCH:
            if j + 1 >= NBUF:
                scats[j + 1 - NBUF].wait()
            loads[j + 1] = load(j + 1)
        loads[j].wait()
        idx16 = idx_v[pl.ds(j * RCH, RCH)]
        g16 = g_v[pl.ds(j * RCH, RCH)]
        if j >= NBUF:
            gscats[j - NBUF].wait()
        for r in range(RCH):
            gstage_v[b, r, pl.ds(0, 16)] = jnp.broadcast_to(g16[r], (16,))
        scats[j] = pltpu.async_copy(rows_v.at[b], disp_hbm.at[idx16],
                                    ssems[b])
        gscats[j] = pltpu.async_copy(gstage_v.at[b], gslot_hbm.at[idx16],
                                     qsems[b])
    for j in range(NCH - NBUF, NCH):
        scats[j].wait()
        gscats[j].wait()


def _expert_kernel(a_ref, w_ref, b_ref, gs_ref, o_ref, abf_ref):
    e = pl.program_id(0)
    n = pl.program_id(1)

    @pl.when((e < E) & (n == 0))
    def _cast():
        abf_ref[...] = a_ref[...].astype(jnp.bfloat16)

    @pl.when(e < E)
    def _compute():
        acc = lax.dot_general(abf_ref[...],
                              w_ref[0].astype(jnp.bfloat16),
                              (((1,), (1,)), ((), ())),
                              preferred_element_type=jnp.float32)
        gs = gs_ref[0][:, 0:1]
        o_ref[0] = (acc + b_ref[0]) * gs

    @pl.when(e == E)
    def _trash():
        o_ref[0] = jnp.zeros((C, NBLK), jnp.float32)


_expert_call = pl.pallas_call(
    _expert_kernel,
    grid=(E + 1, M // NBLK),
    in_specs=[
        pl.BlockSpec((C, M), lambda e, n: (jnp.minimum(e, E - 1), 0)),
        pl.BlockSpec((1, NBLK, M),
                     lambda e, n: (jnp.minimum(e, E - 1), n, 0)),
        pl.BlockSpec((1, 1, NBLK), lambda e, n: (jnp.minimum(e, E - 1), 0, n)),
        pl.BlockSpec((1, C, 128), lambda e, n: (jnp.minimum(e, E - 1), 0, 0)),
    ],
    out_specs=pl.BlockSpec((1, C, NBLK), lambda e, n: (e, 0, n)),
    out_shape=jax.ShapeDtypeStruct((E + 1, C, M), jnp.float32),
    scratch_shapes=[pltpu.VMEM((C, M), jnp.bfloat16)],
    compiler_params=pltpu.CompilerParams(
        dimension_semantics=("arbitrary", "arbitrary")),
)


@functools.partial(
    pl.kernel,
    mesh=_sc_mesh,
    out_type=jax.ShapeDtypeStruct((N, M), jnp.float32),
    scratch_types=[
        pltpu.VMEM((TPW,), jnp.int32),
        pltpu.VMEM((NBUF, RCH, M), jnp.float32),
        pltpu.SemaphoreType.DMA,
        pltpu.SemaphoreType.DMA,
        pltpu.SemaphoreType.DMA,
        pltpu.SemaphoreType.DMA,
        pltpu.SemaphoreType.DMA,
        pltpu.SemaphoreType.DMA,
    ],
)
def _combine(eflat_hbm, sidx_hbm, out_hbm, idx_v, rows_v,
             g0, g1, g2, s0, s1, s2):
    wid = lax.axis_index("s") * 2 + lax.axis_index("c")
    base = wid * TPW
    pltpu.sync_copy(sidx_hbm.at[pl.ds(base, TPW)], idx_v)
    gsems, ssems = [g0, g1, g2], [s0, s1, s2]
    NCH = TPW // RCH

    def gather(j):
        idx16 = idx_v[pl.ds(j * RCH, RCH)]
        return pltpu.async_copy(eflat_hbm.at[idx16], rows_v.at[j % NBUF],
                                gsems[j % NBUF])

    gats = [gather(0)] + [None] * (NCH - 1)
    stores = [None] * NCH
    for j in range(NCH):
        b = j % NBUF
        if j + 1 < NCH:
            if j + 1 >= NBUF:
                stores[j + 1 - NBUF].wait()
            gats[j + 1] = gather(j + 1)
        gats[j].wait()
        stores[j] = pltpu.async_copy(
            rows_v.at[b], out_hbm.at[pl.ds(base + j * RCH, RCH)], ssems[b])
    for j in range(NCH - NBUF, NCH):
        stores[j].wait()


def kernel(x, wg, expert_W, expert_b):
    tokens = x.reshape(N, M)
    sidx3, g3, laux = _gate_call(tokens, wg.T)
    sidx = sidx3.reshape(N)
    g = g3.reshape(N)
    disp, gslot = _dispatch(tokens, sidx, g)
    eout = _expert_call(disp, expert_W, expert_b.reshape(E, 1, M),
                        gslot.reshape(E + 1, C, 128))
    out = _combine(eout.reshape(DISP_ROWS, M), sidx)
    return out.reshape(x.shape), laux.reshape(())


# speedup vs baseline: 1.0009x; 1.0009x over previous
import functools

import jax
import jax.numpy as jnp
from jax import lax
from jax.experimental import pallas as pl
from jax.experimental.pallas import tpu as pltpu
from jax.experimental.pallas import tpu_sc as plsc

N = 8192
M = 2048
E = 8
C = 1024
TRASH = N
DISP_ROWS = (E + 1) * C

NW = 32
TPW = N // NW
RCH = 16
NBUF = 2

TB = 1024
GRID_G = N // TB
NBLK = 1024


def _gate_kernel(tok_ref, wgt_ref, sidx_ref, g_ref, laux_ref,
                 counts_ref, me_ref, tril_ref):
    i = pl.program_id(0)

    @pl.when(i == 0)
    def _init():
        counts_ref[...] = jnp.zeros_like(counts_ref)
        me_ref[...] = jnp.zeros_like(me_ref)
        r_i = lax.broadcasted_iota(jnp.int32, (TB, TB), 0)
        c_i = lax.broadcasted_iota(jnp.int32, (TB, TB), 1)
        tril_ref[...] = (r_i >= c_i).astype(jnp.float32)

    tok = tok_ref[...]
    logits = jnp.dot(tok, wgt_ref[...],
                     preferred_element_type=jnp.float32)
    m = jnp.max(logits, axis=1, keepdims=True)
    ex = jnp.exp(logits - m)
    s = jnp.sum(ex, axis=1, keepdims=True)
    gates = ex / s
    iota_e = lax.broadcasted_iota(jnp.int32, (TB, E), 1)
    idxv = jnp.min(jnp.where(logits == m, iota_e, E), axis=1, keepdims=True)
    mask1 = (iota_e == idxv).astype(jnp.float32)
    gates1 = jnp.sum(gates * mask1, axis=1, keepdims=True)

    loc_incl = jnp.dot(tril_ref[...], mask1,
                       preferred_element_type=jnp.float32)
    loc = loc_incl - 1.0 + counts_ref[...]
    loc_tok = jnp.sum(loc * mask1, axis=1, keepdims=True).astype(jnp.int32)

    keep = loc_tok < C
    flat = idxv * C + loc_tok
    sidx_ref[0] = jnp.where(keep, flat, TRASH)
    g_ref[0] = jnp.where(keep, gates1, 0.0)

    counts_ref[...] += jnp.sum(mask1, axis=0, keepdims=True)
    me_ref[...] += jnp.sum(gates, axis=0, keepdims=True)

    @pl.when(i == GRID_G - 1)
    def _fin():
        laux_ref[...] = (jnp.sum(me_ref[...] * counts_ref[...],
                                 keepdims=True)
                         * (E / (N * N)))


_gate_call = pl.pallas_call(
    _gate_kernel,
    grid=(GRID_G,),
    in_specs=[
        pl.BlockSpec((TB, M), lambda i: (i, 0)),
        pl.BlockSpec((M, E), lambda i: (0, 0)),
    ],
    out_specs=[
        pl.BlockSpec((1, TB, 1), lambda i: (i, 0, 0)),
        pl.BlockSpec((1, TB, 1), lambda i: (i, 0, 0)),
        pl.BlockSpec((1, 1), lambda i: (0, 0)),
    ],
    out_shape=[
        jax.ShapeDtypeStruct((GRID_G, TB, 1), jnp.int32),
        jax.ShapeDtypeStruct((GRID_G, TB, 1), jnp.float32),
        jax.ShapeDtypeStruct((1, 1), jnp.float32),
    ],
    scratch_shapes=[
        pltpu.VMEM((1, E), jnp.float32),
        pltpu.VMEM((1, E), jnp.float32),
        pltpu.VMEM((TB, TB), jnp.float32),
    ],
    compiler_params=pltpu.CompilerParams(
        dimension_semantics=("arbitrary",)),
)


_sc_mesh = plsc.VectorSubcoreMesh(core_axis_name="c", subcore_axis_name="s")


@functools.partial(
    pl.kernel,
    mesh=_sc_mesh,
    out_type=[
        jax.ShapeDtypeStruct((DISP_ROWS, M), jnp.float32),
        jax.ShapeDtypeStruct((DISP_ROWS, 128), jnp.float32),
    ],
    scratch_types=[
        pltpu.VMEM((TPW,), jnp.int32),
        pltpu.VMEM((TPW,), jnp.float32),
        pltpu.VMEM((NBUF, RCH, M), jnp.float32),
        pltpu.VMEM((TPW, 128), jnp.float32),
        pltpu.VMEM((2, 128), jnp.int32),
        pltpu.SemaphoreType.DMA,
        pltpu.SemaphoreType.DMA,
        pltpu.SemaphoreType.DMA,
        pltpu.SemaphoreType.DMA,
        pltpu.SemaphoreType.DMA,
        pltpu.SemaphoreType.DMA,
    ],
)
def _dispatch(tok_hbm, sidx_hbm, g_hbm, disp_hbm, gslot_hbm,
              idx_v, g_v, rows_v, gstage_v, idx2_v, l0, l1, s0, s1, q0, q1):
    wid = lax.axis_index("s") * 2 + lax.axis_index("c")
    base = wid * TPW
    pltpu.sync_copy(sidx_hbm.at[pl.ds(base, TPW)], idx_v)
    pltpu.sync_copy(g_hbm.at[pl.ds(base, TPW)], g_v)
    pltpu.sync_copy(sidx_hbm.at[pl.ds(base, 128)], idx2_v.at[0])
    pltpu.sync_copy(sidx_hbm.at[pl.ds(base + 128, 128)], idx2_v.at[1])
    lsems, ssems, qsems = [l0, l1], [s0, s1], [q0, q1]
    NCH = TPW // RCH

    def load(j):
        return pltpu.async_copy(tok_hbm.at[pl.ds(base + j * RCH, RCH)],
                                rows_v.at[j % NBUF], lsems[j % NBUF])

    loads = [load(0)] + [None] * (NCH - 1)
    scats = [None] * NCH
    for j in range(NCH):
        b = j % NBUF
        if j + 1 < NCH:
            if j + 1 >= NBUF:
                scats[j + 1 - NBUF].wait()
            loads[j + 1] = load(j + 1)
        loads[j].wait()
        idx16 = idx_v[pl.ds(j * RCH, RCH)]
        g16 = g_v[pl.ds(j * RCH, RCH)]
        for r in range(RCH):
            gstage_v[j * RCH + r, pl.ds(0, 16)] = jnp.broadcast_to(
                g16[r], (16,))
        scats[j] = pltpu.async_copy(rows_v.at[b], disp_hbm.at[idx16],
                                    ssems[b])
    gq0 = pltpu.async_copy(gstage_v.at[pl.ds(0, 128)],
                           gslot_hbm.at[idx2_v.at[0]], qsems[0])
    gq1 = pltpu.async_copy(gstage_v.at[pl.ds(128, 128)],
                           gslot_hbm.at[idx2_v.at[1]], qsems[1])
    for j in range(NCH - NBUF, NCH):
        scats[j].wait()
    gq0.wait()
    gq1.wait()


def _expert_kernel(a_ref, w_ref, b_ref, gs_ref, o_ref):
    e = pl.program_id(0)

    @pl.when(e < E)
    def _compute():
        acc = lax.dot_general(a_ref[...].astype(jnp.bfloat16),
                              w_ref[0].astype(jnp.bfloat16),
                              (((1,), (1,)), ((), ())),
                              preferred_element_type=jnp.float32)
        gs = gs_ref[0][:, 0:1]
        o_ref[0] = (acc + b_ref[0]) * gs

    @pl.when(e == E)
    def _trash():
        o_ref[0] = jnp.zeros((C, NBLK), jnp.float32)


_expert_call = pl.pallas_call(
    _expert_kernel,
    grid=(E + 1, M // NBLK),
    in_specs=[
        pl.BlockSpec((C, M), lambda e, n: (jnp.minimum(e, E - 1), 0)),
        pl.BlockSpec((1, NBLK, M),
                     lambda e, n: (jnp.minimum(e, E - 1), n, 0)),
        pl.BlockSpec((1, 1, NBLK), lambda e, n: (jnp.minimum(e, E - 1), 0, n)),
        pl.BlockSpec((1, C, 128), lambda e, n: (jnp.minimum(e, E - 1), 0, 0)),
    ],
    out_specs=pl.BlockSpec((1, C, NBLK), lambda e, n: (e, 0, n)),
    out_shape=jax.ShapeDtypeStruct((E + 1, C, M), jnp.float32),
    compiler_params=pltpu.CompilerParams(
        dimension_semantics=("arbitrary", "arbitrary")),
)


@functools.partial(
    pl.kernel,
    mesh=_sc_mesh,
    out_type=jax.ShapeDtypeStruct((N, M), jnp.float32),
    scratch_types=[
        pltpu.VMEM((TPW,), jnp.int32),
        pltpu.VMEM((NBUF, RCH, M), jnp.float32),
        pltpu.SemaphoreType.DMA,
        pltpu.SemaphoreType.DMA,
        pltpu.SemaphoreType.DMA,
        pltpu.SemaphoreType.DMA,
    ],
)
def _combine(eflat_hbm, sidx_hbm, out_hbm, idx_v, rows_v, g0, g1, s0, s1):
    wid = lax.axis_index("s") * 2 + lax.axis_index("c")
    base = wid * TPW
    pltpu.sync_copy(sidx_hbm.at[pl.ds(base, TPW)], idx_v)
    gsems, ssems = [g0, g1], [s0, s1]
    NCH = TPW // RCH

    def gather(j):
        idx16 = idx_v[pl.ds(j * RCH, RCH)]
        return pltpu.async_copy(eflat_hbm.at[idx16], rows_v.at[j % NBUF],
                                gsems[j % NBUF])

    gats = [gather(0)] + [None] * (NCH - 1)
    stores = [None] * NCH
    for j in range(NCH):
        b = j % NBUF
        if j + 1 < NCH:
            if j + 1 >= NBUF:
                stores[j + 1 - NBUF].wait()
            gats[j + 1] = gather(j + 1)
        gats[j].wait()
        stores[j] = pltpu.async_copy(
            rows_v.at[b], out_hbm.at[pl.ds(base + j * RCH, RCH)], ssems[b])
    for j in range(NCH - NBUF, NCH):
        stores[j].wait()


def kernel(x, wg, expert_W, expert_b):
    tokens = x.reshape(N, M)
    sidx3, g3, laux = _gate_call(tokens, wg.T)
    sidx = sidx3.reshape(N)
    g = g3.reshape(N)
    disp, gslot = _dispatch(tokens, sidx, g)
    eout = _expert_call(disp, expert_W, expert_b.reshape(E, 1, M),
                        gslot.reshape(E + 1, C, 128))
    out = _combine(eout.reshape(DISP_ROWS, M), sidx)
    return out.reshape(x.shape), laux.reshape(())


# speedup vs baseline: 1.0079x; 1.0069x over previous
import functools

import jax
import jax.numpy as jnp
from jax import lax
from jax.experimental import pallas as pl
from jax.experimental.pallas import tpu as pltpu
from jax.experimental.pallas import tpu_sc as plsc

N = 8192
M = 2048
E = 8
C = 1024
TRASH = N
DISP_ROWS = (E + 1) * C

NW = 32
TPW = N // NW
RCH = 16
NBUF = 2

TB = 1024
GRID_G = N // TB
NBLK = 1024


def _gate_kernel(tok_ref, wgt_ref, sidx_ref, g_ref, laux_ref,
                 counts_ref, me_ref, tril_ref):
    i = pl.program_id(0)

    @pl.when(i == 0)
    def _init():
        counts_ref[...] = jnp.zeros_like(counts_ref)
        me_ref[...] = jnp.zeros_like(me_ref)
        r_i = lax.broadcasted_iota(jnp.int32, (TB, TB), 0)
        c_i = lax.broadcasted_iota(jnp.int32, (TB, TB), 1)
        tril_ref[...] = (r_i >= c_i).astype(jnp.float32)

    tok = tok_ref[...]
    logits = jnp.dot(tok, wgt_ref[...],
                     preferred_element_type=jnp.float32)
    m = jnp.max(logits, axis=1, keepdims=True)
    ex = jnp.exp(logits - m)
    s = jnp.sum(ex, axis=1, keepdims=True)
    gates = ex / s
    iota_e = lax.broadcasted_iota(jnp.int32, (TB, E), 1)
    idxv = jnp.min(jnp.where(logits == m, iota_e, E), axis=1, keepdims=True)
    mask1 = (iota_e == idxv).astype(jnp.float32)
    gates1 = jnp.sum(gates * mask1, axis=1, keepdims=True)

    loc_incl = jnp.dot(tril_ref[...], mask1,
                       preferred_element_type=jnp.float32)
    loc = loc_incl - 1.0 + counts_ref[...]
    loc_tok = jnp.sum(loc * mask1, axis=1, keepdims=True).astype(jnp.int32)

    keep = loc_tok < C
    flat = idxv * C + loc_tok
    sidx_ref[0] = jnp.where(keep, flat, TRASH)
    g_ref[0] = jnp.where(keep, gates1, 0.0)

    counts_ref[...] += jnp.sum(mask1, axis=0, keepdims=True)
    me_ref[...] += jnp.sum(gates, axis=0, keepdims=True)

    @pl.when(i == GRID_G - 1)
    def _fin():
        laux_ref[...] = (jnp.sum(me_ref[...] * counts_ref[...],
                                 keepdims=True)
                         * (E / (N * N)))


_gate_call = pl.pallas_call(
    _gate_kernel,
    grid=(GRID_G,),
    in_specs=[
        pl.BlockSpec((TB, M), lambda i: (i, 0)),
        pl.BlockSpec((M, E), lambda i: (0, 0)),
    ],
    out_specs=[
        pl.BlockSpec((1, TB, 1), lambda i: (i, 0, 0)),
        pl.BlockSpec((1, TB, 1), lambda i: (i, 0, 0)),
        pl.BlockSpec((1, 1), lambda i: (0, 0)),
    ],
    out_shape=[
        jax.ShapeDtypeStruct((GRID_G, TB, 1), jnp.int32),
        jax.ShapeDtypeStruct((GRID_G, TB, 1), jnp.float32),
        jax.ShapeDtypeStruct((1, 1), jnp.float32),
    ],
    scratch_shapes=[
        pltpu.VMEM((1, E), jnp.float32),
        pltpu.VMEM((1, E), jnp.float32),
        pltpu.VMEM((TB, TB), jnp.float32),
    ],
    compiler_params=pltpu.CompilerParams(
        dimension_semantics=("arbitrary",)),
)


_sc_mesh = plsc.VectorSubcoreMesh(core_axis_name="c", subcore_axis_name="s")


@functools.partial(
    pl.kernel,
    mesh=_sc_mesh,
    out_type=[
        jax.ShapeDtypeStruct((DISP_ROWS, M), jnp.float32),
        jax.ShapeDtypeStruct((DISP_ROWS, 128), jnp.float32),
    ],
    scratch_types=[
        pltpu.VMEM((TPW,), jnp.int32),
        pltpu.VMEM((TPW,), jnp.float32),
        pltpu.VMEM((NBUF, RCH, M), jnp.float32),
        pltpu.VMEM((NBUF, RCH, 128), jnp.float32),
        pltpu.SemaphoreType.DMA,
        pltpu.SemaphoreType.DMA,
        pltpu.SemaphoreType.DMA,
        pltpu.SemaphoreType.DMA,
        pltpu.SemaphoreType.DMA,
        pltpu.SemaphoreType.DMA,
    ],
)
def _dispatch(tok_hbm, sidx_hbm, g_hbm, disp_hbm, gslot_hbm,
              idx_v, g_v, rows_v, gstage_v, l0, l1, s0, s1, q0, q1):
    wid = lax.axis_index("s") * 2 + lax.axis_index("c")
    base = wid * TPW
    pltpu.sync_copy(sidx_hbm.at[pl.ds(base, TPW)], idx_v)
    pltpu.sync_copy(g_hbm.at[pl.ds(base, TPW)], g_v)
    lsems, ssems, qsems = [l0, l1], [s0, s1], [q0, q1]
    NCH = TPW // RCH

    def load(j):
        return pltpu.async_copy(tok_hbm.at[pl.ds(base + j * RCH, RCH)],
                                rows_v.at[j % NBUF], lsems[j % NBUF])

    loads = [load(0)] + [None] * (NCH - 1)
    scats = [None] * NCH
    gscats = [None] * NCH
    for j in range(NCH):
        b = j % NBUF
        if j + 1 < NCH:
            if j + 1 >= NBUF:
                scats[j + 1 - NBUF].wait()
            loads[j + 1] = load(j + 1)
        loads[j].wait()
        idx16 = idx_v[pl.ds(j * RCH, RCH)]
        g16 = g_v[pl.ds(j * RCH, RCH)]
        if j >= NBUF:
            gscats[j - NBUF].wait()
        for r in range(RCH):
            gstage_v[b, r, pl.ds(0, 16)] = jnp.broadcast_to(g16[r], (16,))
        scats[j] = pltpu.async_copy(rows_v.at[b], disp_hbm.at[idx16],
                                    ssems[b])
        gscats[j] = pltpu.async_copy(gstage_v.at[b], gslot_hbm.at[idx16],
                                     qsems[b])
    for j in range(NCH - NBUF, NCH):
        scats[j].wait()
        gscats[j].wait()


def _expert_kernel(a_ref, w_ref, b_ref, gs_ref, o_ref):
    e = pl.program_id(0)

    @pl.when(e < E)
    def _compute():
        acc = lax.dot_general(a_ref[...], w_ref[0],
                              (((1,), (1,)), ((), ())),
                              preferred_element_type=jnp.float32)
        gs = gs_ref[0][:, 0:1]
        o_ref[0] = (acc + b_ref[0]) * gs

    @pl.when(e == E)
    def _trash():
        o_ref[0] = jnp.zeros((C, NBLK), jnp.float32)


_expert_call = pl.pallas_call(
    _expert_kernel,
    grid=(E + 1, M // NBLK),
    in_specs=[
        pl.BlockSpec((C, M), lambda e, n: (jnp.minimum(e, E - 1), 0)),
        pl.BlockSpec((1, NBLK, M),
                     lambda e, n: (jnp.minimum(e, E - 1), n, 0)),
        pl.BlockSpec((1, 1, NBLK), lambda e, n: (jnp.minimum(e, E - 1), 0, n)),
        pl.BlockSpec((1, C, 128), lambda e, n: (jnp.minimum(e, E - 1), 0, 0)),
    ],
    out_specs=pl.BlockSpec((1, C, NBLK), lambda e, n: (e, 0, n)),
    out_shape=jax.ShapeDtypeStruct((E + 1, C, M), jnp.float32),
    compiler_params=pltpu.CompilerParams(
        dimension_semantics=("arbitrary", "arbitrary")),
)


@functools.partial(
    pl.kernel,
    mesh=_sc_mesh,
    out_type=jax.ShapeDtypeStruct((N, M), jnp.float32),
    scratch_types=[
        pltpu.VMEM((TPW,), jnp.int32),
        pltpu.VMEM((NBUF, RCH, M), jnp.float32),
        pltpu.SemaphoreType.DMA,
        pltpu.SemaphoreType.DMA,
        pltpu.SemaphoreType.DMA,
        pltpu.SemaphoreType.DMA,
    ],
)
def _combine(eflat_hbm, sidx_hbm, out_hbm, idx_v, rows_v, g0, g1, s0, s1):
    wid = lax.axis_index("s") * 2 + lax.axis_index("c")
    base = wid * TPW
    pltpu.sync_copy(sidx_hbm.at[pl.ds(base, TPW)], idx_v)
    gsems, ssems = [g0, g1], [s0, s1]
    NCH = TPW // RCH

    def gather(j):
        idx16 = idx_v[pl.ds(j * RCH, RCH)]
        return pltpu.async_copy(eflat_hbm.at[idx16], rows_v.at[j % NBUF],
                                gsems[j % NBUF])

    gats = [gather(0)] + [None] * (NCH - 1)
    stores = [None] * NCH
    for j in range(NCH):
        b = j % NBUF
        if j + 1 < NCH:
            if j + 1 >= NBUF:
                stores[j + 1 - NBUF].wait()
            gats[j + 1] = gather(j + 1)
        gats[j].wait()
        stores[j] = pltpu.async_copy(
            rows_v.at[b], out_hbm.at[pl.ds(base + j * RCH, RCH)], ssems[b])
    for j in range(NCH - NBUF, NCH):
        stores[j].wait()


def kernel(x, wg, expert_W, expert_b):
    tokens = x.reshape(N, M)
    sidx3, g3, laux = _gate_call(tokens, wg.T)
    sidx = sidx3.reshape(N)
    g = g3.reshape(N)
    disp, gslot = _dispatch(tokens, sidx, g)
    eout = _expert_call(disp, expert_W, expert_b.reshape(E, 1, M),
                        gslot.reshape(E + 1, C, 128))
    out = _combine(eout.reshape(DISP_ROWS, M), sidx)
    return out.reshape(x.shape), laux.reshape(())


# speedup vs baseline: 1.0079x; 1.0000x over previous
import functools

import jax
import jax.numpy as jnp
from jax import lax
from jax.experimental import pallas as pl
from jax.experimental.pallas import tpu as pltpu
from jax.experimental.pallas import tpu_sc as plsc

N = 8192
M = 2048
E = 8
C = 1024
TRASH = N
DISP_ROWS = (E + 1) * C

NW = 32
TPW = N // NW
RCH = 16
NBUF = 2

TB = 1024
GRID_G = N // TB
NBLK = 1024


def _gate_kernel(tok_ref, wgt_ref, sidx_ref, g_ref, laux_ref,
                 counts_ref, me_ref, tril_ref):
    i = pl.program_id(0)

    @pl.when(i == 0)
    def _init():
        counts_ref[...] = jnp.zeros_like(counts_ref)
        me_ref[...] = jnp.zeros_like(me_ref)
        r_i = lax.broadcasted_iota(jnp.int32, (TB, TB), 0)
        c_i = lax.broadcasted_iota(jnp.int32, (TB, TB), 1)
        tril_ref[...] = (r_i >= c_i).astype(jnp.float32)

    tok = tok_ref[...]
    logits = jnp.dot(tok, wgt_ref[...],
                     preferred_element_type=jnp.float32)
    m = jnp.max(logits, axis=1, keepdims=True)
    ex = jnp.exp(logits - m)
    s = jnp.sum(ex, axis=1, keepdims=True)
    gates = ex / s
    iota_e = lax.broadcasted_iota(jnp.int32, (TB, E), 1)
    idxv = jnp.min(jnp.where(logits == m, iota_e, E), axis=1, keepdims=True)
    mask1 = (iota_e == idxv).astype(jnp.float32)
    gates1 = jnp.sum(gates * mask1, axis=1, keepdims=True)

    loc_incl = jnp.dot(tril_ref[...], mask1,
                       preferred_element_type=jnp.float32)
    loc = loc_incl - 1.0 + counts_ref[...]
    loc_tok = jnp.sum(loc * mask1, axis=1, keepdims=True).astype(jnp.int32)

    keep = loc_tok < C
    flat = idxv * C + loc_tok
    sidx_ref[0] = jnp.where(keep, flat, TRASH)
    g_ref[0] = jnp.where(keep, gates1, 0.0)

    counts_ref[...] += jnp.sum(mask1, axis=0, keepdims=True)
    me_ref[...] += jnp.sum(gates, axis=0, keepdims=True)

    @pl.when(i == GRID_G - 1)
    def _fin():
        laux_ref[...] = (jnp.sum(me_ref[...] * counts_ref[...],
                                 keepdims=True)
                         * (E / (N * N)))


_gate_call = pl.pallas_call(
    _gate_kernel,
    grid=(GRID_G,),
    in_specs=[
        pl.BlockSpec((TB, M), lambda i: (i, 0)),
        pl.BlockSpec((M, E), lambda i: (0, 0)),
    ],
    out_specs=[
        pl.BlockSpec((1, TB, 1), lambda i: (i, 0, 0)),
        pl.BlockSpec((1, TB, 1), lambda i: (i, 0, 0)),
        pl.BlockSpec((1, 1), lambda i: (0, 0)),
    ],
    out_shape=[
        jax.ShapeDtypeStruct((GRID_G, TB, 1), jnp.int32),
        jax.ShapeDtypeStruct((GRID_G, TB, 1), jnp.float32),
        jax.ShapeDtypeStruct((1, 1), jnp.float32),
    ],
    scratch_shapes=[
        pltpu.VMEM((1, E), jnp.float32),
        pltpu.VMEM((1, E), jnp.float32),
        pltpu.VMEM((TB, TB), jnp.float32),
    ],
    compiler_params=pltpu.CompilerParams(
        dimension_semantics=("arbitrary",)),
)


_sc_mesh = plsc.VectorSubcoreMesh(core_axis_name="c", subcore_axis_name="s")


@functools.partial(
    pl.kernel,
    mesh=_sc_mesh,
    out_type=[
        jax.ShapeDtypeStruct((DISP_ROWS, M), jnp.float32),
        jax.ShapeDtypeStruct((DISP_ROWS, 128), jnp.float32),
    ],
    scratch_types=[
        pltpu.VMEM((TPW,), jnp.int32),
        pltpu.VMEM((TPW,), jnp.float32),
        pltpu.VMEM((NBUF, RCH, M), jnp.float32),
        pltpu.VMEM((NBUF, RCH, 128), jnp.float32),
        pltpu.SemaphoreType.DMA,
        pltpu.SemaphoreType.DMA,
        pltpu.SemaphoreType.DMA,
        pltpu.SemaphoreType.DMA,
        pltpu.SemaphoreType.DMA,
        pltpu.SemaphoreType.DMA,
    ],
)
def _dispatch(tok_hbm, sidx_hbm, g_hbm, disp_hbm, gslot_hbm,
              idx_v, g_v, rows_v, gstage_v, l0, l1, s0, s1, q0, q1):
    wid = lax.axis_index("s") * 2 + lax.axis_index("c")
    base = wid * TPW
    pltpu.sync_copy(sidx_hbm.at[pl.ds(base, TPW)], idx_v)
    pltpu.sync_copy(g_hbm.at[pl.ds(base, TPW)], g_v)
    lsems, ssems, qsems = [l0, l1], [s0, s1], [q0, q1]
    NCH = TPW // RCH

    def load(j):
        return pltpu.async_copy(tok_hbm.at[pl.ds(base + j * RCH, RCH)],
                                rows_v.at[j % NBUF], lsems[j % NBUF])

    loads = [load(0)] + [None] * (NCH - 1)
    scats = [None] * NCH
    gscats = [None] * NCH
    for j in range(NCH):
        b = j % NBUF
        if j + 1 < NCH:
            if j + 1 >= NBUF:
                scats[j + 1 - NBUF].wait()
            loads[j + 1] = load(j + 1)
        loads[j].wait()
        idx16 = idx_v[pl.ds(j * RCH, RCH)]
        g16 = g_v[pl.ds(j * RCH, RCH)]
        if j >= NBUF:
            gscats[j - NBUF].wait()
        for r in range(RCH):
            gstage_v[b, r, pl.ds(0, 16)] = jnp.broadcast_to(g16[r], (16,))
        scats[j] = pltpu.async_copy(rows_v.at[b], disp_hbm.at[idx16],
                                    ssems[b])
        gscats[j] = pltpu.async_copy(gstage_v.at[b], gslot_hbm.at[idx16],
                                     qsems[b])
    for j in range(NCH - NBUF, NCH):
        scats[j].wait()
        gscats[j].wait()


def _expert_kernel(a_ref, w1_ref, w2_ref, b_ref, gs_ref, o_ref):
    e = pl.program_id(0)

    @pl.when(e < E)
    def _compute():
        dn = (((1,), (1,)), ((), ()))
        acc = lax.dot_general(a_ref[:, : M // 2], w1_ref[0], dn,
                              preferred_element_type=jnp.float32)
        acc = acc + lax.dot_general(a_ref[:, M // 2:], w2_ref[0], dn,
                                    preferred_element_type=jnp.float32)
        gs = gs_ref[0][:, 0:1]
        o_ref[0] = (acc + b_ref[0]) * gs

    @pl.when(e == E)
    def _trash():
        o_ref[0] = jnp.zeros((C, NBLK), jnp.float32)


_expert_call = pl.pallas_call(
    _expert_kernel,
    grid=(E + 1, M // NBLK),
    in_specs=[
        pl.BlockSpec((C, M), lambda e, n: (jnp.minimum(e, E - 1), 0)),
        pl.BlockSpec((1, NBLK, M // 2),
                     lambda e, n: (jnp.minimum(e, E - 1), n, 0)),
        pl.BlockSpec((1, NBLK, M // 2),
                     lambda e, n: (jnp.minimum(e, E - 1), n, 1)),
        pl.BlockSpec((1, 1, NBLK), lambda e, n: (jnp.minimum(e, E - 1), 0, n)),
        pl.BlockSpec((1, C, 128), lambda e, n: (jnp.minimum(e, E - 1), 0, 0)),
    ],
    out_specs=pl.BlockSpec((1, C, NBLK), lambda e, n: (e, 0, n)),
    out_shape=jax.ShapeDtypeStruct((E + 1, C, M), jnp.float32),
    compiler_params=pltpu.CompilerParams(
        dimension_semantics=("arbitrary", "arbitrary")),
)


@functools.partial(
    pl.kernel,
    mesh=_sc_mesh,
    out_type=jax.ShapeDtypeStruct((N, M), jnp.float32),
    scratch_types=[
        pltpu.VMEM((TPW,), jnp.int32),
        pltpu.VMEM((NBUF, RCH, M), jnp.float32),
        pltpu.SemaphoreType.DMA,
        pltpu.SemaphoreType.DMA,
        pltpu.SemaphoreType.DMA,
        pltpu.SemaphoreType.DMA,
    ],
)
def _combine(eflat_hbm, sidx_hbm, out_hbm, idx_v, rows_v, g0, g1, s0, s1):
    wid = lax.axis_index("s") * 2 + lax.axis_index("c")
    base = wid * TPW
    pltpu.sync_copy(sidx_hbm.at[pl.ds(base, TPW)], idx_v)
    gsems, ssems = [g0, g1], [s0, s1]
    NCH = TPW // RCH

    def gather(j):
        idx16 = idx_v[pl.ds(j * RCH, RCH)]
        return pltpu.async_copy(eflat_hbm.at[idx16], rows_v.at[j % NBUF],
                                gsems[j % NBUF])

    gats = [gather(0)] + [None] * (NCH - 1)
    stores = [None] * NCH
    for j in range(NCH):
        b = j % NBUF
        if j + 1 < NCH:
            if j + 1 >= NBUF:
                stores[j + 1 - NBUF].wait()
            gats[j + 1] = gather(j + 1)
        gats[j].wait()
        stores[j] = pltpu.async_copy(
            rows_v.at[b], out_hbm.at[pl.ds(base + j * RCH, RCH)], ssems[b])
    for j in range(NCH - NBUF, NCH):
        stores[j].wait()


def kernel(x, wg, expert_W, expert_b):
    tokens = x.reshape(N, M)
    sidx3, g3, laux = _gate_call(tokens, wg.T)
    sidx = sidx3.reshape(N)
    g = g3.reshape(N)
    disp, gslot = _dispatch(tokens, sidx, g)
    eout = _expert_call(disp, expert_W, expert_W,
                        expert_b.reshape(E, 1, M),
                        gslot.reshape(E + 1, C, 128))
    out = _combine(eout.reshape(DISP_ROWS, M), sidx)
    return out.reshape(x.shape), laux.reshape(())
